# baseline (speedup 1.0000x reference)
import functools

import jax
import jax.numpy as jnp
from jax import lax
from jax.experimental import pallas as pl
from jax.experimental.pallas import tpu as pltpu
from jax.experimental.pallas import tpu_sc as plsc

_B, _S, _H = 4, 8192, 768
_E, _TOPK = 8, 2
_N = _B * _S

_BLOCK_T = 4096
_GRID = _N // _BLOCK_T

_NTILES = 32
_TPT = _N // _NTILES
_L = 16


def _gate_kernel(x_ref, w_ref, out_ref):
    out_ref[...] = jax.lax.dot_general(
        w_ref[...], x_ref[...], (((1,), (1,)), ((), ())),
        preferred_element_type=jnp.float32)


def _sc_router(lgt_hbm, probs_hbm, wts_hbm, idx_hbm, ewp_hbm, ecp_hbm,
               lg_v, pb_v, wb_v, ib_v, ew_v, ec_v):
    wid = lax.axis_index("s") * 2 + lax.axis_index("c")
    base = wid * _TPT
    pltpu.sync_copy(lgt_hbm.at[:, pl.ds(base, _TPT)], lg_v)

    zero = jnp.zeros((_L,), jnp.float32)
    one = jnp.ones((_L,), jnp.float32)
    ew_v[...] = zero
    ec_v[...] = zero

    def chunk(c, carry):
        off = c * _L
        ls = [lg_v[e, pl.ds(off, _L)] for e in range(_E)]
        m = ls[0]
        for e in range(1, _E):
            m = jnp.maximum(m, ls[e])
        exs = [jnp.exp(l - m) for l in ls]
        s = exs[0]
        for e in range(1, _E):
            s = s + exs[e]
        inv = 1.0 / s
        ps = [ex * inv for ex in exs]

        best = ps[0]
        bidx = jnp.zeros((_L,), jnp.int32)
        for e in range(1, _E):
            upd = ps[e] > best
            best = jnp.where(upd, ps[e], best)
            bidx = jnp.where(upd, e, bidx)
        second = jnp.full((_L,), -1.0, jnp.float32)
        sidx = jnp.zeros((_L,), jnp.int32)
        for e in range(_E):
            upd = (ps[e] > second) & (bidx != e)
            second = jnp.where(upd, ps[e], second)
            sidx = jnp.where(upd, e, sidx)

        inv2 = 1.0 / (best + second + 1e-8)
        w1 = best * inv2
        w2 = second * inv2

        for e in range(_E):
            pb_v[e, pl.ds(off, _L)] = ps[e]
        wb_v[0, pl.ds(off, _L)] = w1
        wb_v[1, pl.ds(off, _L)] = w2
        ib_v[0, pl.ds(off, _L)] = bidx
        ib_v[1, pl.ds(off, _L)] = sidx

        plsc.addupdate_scatter(ew_v, [bidx], w1)
        plsc.addupdate_scatter(ew_v, [sidx], w2)
        plsc.addupdate_scatter(ec_v, [bidx], one)
        plsc.addupdate_scatter(ec_v, [sidx], one)
        return carry

    lax.fori_loop(0, _TPT // _L, chunk, 0)

    pltpu.sync_copy(pb_v, probs_hbm.at[:, pl.ds(base, _TPT)])
    pltpu.sync_copy(wb_v, wts_hbm.at[:, pl.ds(base, _TPT)])
    pltpu.sync_copy(ib_v, idx_hbm.at[:, pl.ds(base, _TPT)])
    pltpu.sync_copy(ew_v, ewp_hbm.at[wid])
    pltpu.sync_copy(ec_v, ecp_hbm.at[wid])


_scmesh = plsc.VectorSubcoreMesh(core_axis_name="c", subcore_axis_name="s")

_sc_router_call = functools.partial(
    pl.kernel,
    mesh=_scmesh,
    out_type=[
        jax.ShapeDtypeStruct((_E, _N), jnp.float32),
        jax.ShapeDtypeStruct((_TOPK, _N), jnp.float32),
        jax.ShapeDtypeStruct((_TOPK, _N), jnp.int32),
        jax.ShapeDtypeStruct((_NTILES, _L), jnp.float32),
        jax.ShapeDtypeStruct((_NTILES, _L), jnp.float32),
    ],
    scratch_types=[
        pltpu.VMEM((_E, _TPT), jnp.float32),
        pltpu.VMEM((_E, _TPT), jnp.float32),
        pltpu.VMEM((_TOPK, _TPT), jnp.float32),
        pltpu.VMEM((_TOPK, _TPT), jnp.int32),
        pltpu.VMEM((_L,), jnp.float32),
        pltpu.VMEM((_L,), jnp.float32),
    ],
    compiler_params=pltpu.CompilerParams(needs_layout_passes=False),
)(_sc_router)


def kernel(hidden_states, W):
    x = hidden_states.reshape(_N, _H)
    lgt = pl.pallas_call(
        _gate_kernel,
        grid=(_GRID,),
        in_specs=[
            pl.BlockSpec((_BLOCK_T, _H), lambda i: (i, 0)),
            pl.BlockSpec((_E, _H), lambda i: (0, 0)),
        ],
        out_specs=pl.BlockSpec((_E, _BLOCK_T), lambda i: (0, i)),
        out_shape=jax.ShapeDtypeStruct((_E, _N), jnp.float32),
        compiler_params=pltpu.CompilerParams(
            dimension_semantics=("arbitrary",)),
    )(x, W)

    probs_f, wts_f, idx_f, ewp, ecp = _sc_router_call(lgt)

    ew = jnp.sum(ewp, axis=0)
    ec = jnp.sum(ecp, axis=0)
    expected = _N * _TOPK / _E
    loss = jnp.sum(ew * ec) / (expected * expected)

    return (wts_f.T.reshape(_B, _S, _TOPK), idx_f.T.reshape(_B, _S, _TOPK),
            probs_f.T.reshape(_B, _S, _E), loss)
